# baseline (speedup 1.0000x reference)
import functools

import jax
import jax.numpy as jnp
from jax import lax
from jax.experimental import pallas as pl
from jax.experimental.pallas import tpu as pltpu
from jax.experimental.pallas import tpu_sc as plsc

N = 10000
E = 320000
IN = 128
H = 8
D = 8
T = 8

NC = 2
NS = 16
NW = NC * NS
EPW = E // NW
C = 1000
NCHUNK = EPW // C


def _tc_body(x_ref, et_ref, wcat_ref, g32_ref, el_ref, er_ref):
    z = jnp.dot(x_ref[...], wcat_ref[...], preferred_element_type=jnp.float32)
    colt = lax.broadcasted_iota(jnp.int32, z.shape, 1) // 16
    masked = jnp.where(colt == et_ref[...], z, 0.0)
    elr = jnp.dot(masked, g32_ref[...], preferred_element_type=jnp.float32)
    el_ref[...] = elr[:, :16]
    er_ref[...] = elr[:, 16:]


def _tc_tables(x, etype, wcat, g32):
    return pl.pallas_call(
        _tc_body,
        out_shape=(
            jax.ShapeDtypeStruct((N, 16), jnp.float32),
            jax.ShapeDtypeStruct((N, 16), jnp.float32),
        ),
    )(x, etype.reshape(N, 1), wcat, g32)


def _sc_pass1(el16_hbm, er16_hbm, src_hbm, dst_hbm,
              p_hbm, s0_hbm, s1_hbm,
              src_v0, src_v1, dst_v0, dst_v1,
              el_g0, el_g1, er_g0, er_g1,
              p_v, p_lin, s_sh, sem0, sem1, sem2, sem3):
    cid = lax.axis_index("c")
    sid = lax.axis_index("s")
    wid = sid * NC + cid
    base = wid * EPW
    lo = lax.iota(jnp.int32, 16) < 8
    src_v = [src_v0, src_v1]
    dst_v = [dst_v0, dst_v1]
    el_g = [el_g0, el_g1]
    er_g = [er_g0, er_g1]
    sems = [[sem0, sem1], [sem2, sem3]]

    @pl.when(sid == 0)
    def _():
        def zrow(r, _):
            p_v[r] = jnp.zeros((16,), jnp.float32)
            return 0
        lax.fori_loop(0, C, zrow, 0)
        for k in range(N // C):
            pltpu.sync_copy(p_v, s_sh.at[pl.ds(k * C, C)])

    plsc.subcore_barrier()

    copies = [None, None]
    for k in range(NCHUNK + 1):
        if k < NCHUNK:
            b = k % 2
            off = base + k * C
            pltpu.sync_copy(src_hbm.at[pl.ds(off, C)], src_v[b])
            pltpu.sync_copy(dst_hbm.at[pl.ds(off, C)], dst_v[b])
            copies[b] = (
                pltpu.async_copy(el16_hbm.at[src_v[b]], el_g[b], sems[b][0]),
                pltpu.async_copy(er16_hbm.at[dst_v[b]], er_g[b], sems[b][1]),
            )
        if k > 0:
            b = (k - 1) % 2
            off = base + (k - 1) * C
            copies[b][0].wait()
            copies[b][1].wait()
            elg = el_g[b]
            erg = er_g[b]

            @plsc.parallel_loop(0, C // 2, unroll=8)
            def _(q):
                r0 = 2 * q
                r1 = r0 + 1
                v0 = elg[r0] + erg[r0]
                v0 = jnp.where(v0 > 0, v0, 0.2 * v0)
                e0 = jnp.exp(v0)
                v1 = elg[r1] + erg[r1]
                v1 = jnp.where(v1 > 0, v1, 0.2 * v1)
                e1 = jnp.exp(v1)
                p_v[r0] = e0
                p_v[r1] = e1
                p_lin[pl.ds(q * 16, 16)] = jnp.where(lo, e0, e1)

            pltpu.sync_copy(p_lin, p_hbm.at[pl.ds(off * H, C * H)])
            pltpu.sync_copy(p_v, s_sh.at[dst_v[b]], add=True)

    plsc.subcore_barrier()

    @pl.when(jnp.logical_and(sid == 0, cid == 0))
    def _():
        pltpu.sync_copy(s_sh, s0_hbm)

    @pl.when(jnp.logical_and(sid == 0, cid == 1))
    def _():
        pltpu.sync_copy(s_sh, s1_hbm)


def _sc_pass2(p_hbm, dst_hbm, s0_hbm, s1_hbm, a_hbm,
              dst_v0, dst_v1, p_v0, p_v1,
              s0_g0, s0_g1, s1_g0, s1_g1, o_v,
              sem0, sem1, sem2, sem3):
    cid = lax.axis_index("c")
    sid = lax.axis_index("s")
    wid = sid * NC + cid
    base = wid * EPW
    lo = lax.iota(jnp.int32, 16) < 8
    dst_v = [dst_v0, dst_v1]
    p_v = [p_v0, p_v1]
    s0_g = [s0_g0, s0_g1]
    s1_g = [s1_g0, s1_g1]
    sems = [[sem0, sem1], [sem2, sem3]]

    copies = [None, None]
    for k in range(NCHUNK + 1):
        if k < NCHUNK:
            b = k % 2
            off = base + k * C
            pltpu.sync_copy(dst_hbm.at[pl.ds(off, C)], dst_v[b])
            pltpu.sync_copy(p_hbm.at[pl.ds(off * H, C * H)], p_v[b])
            copies[b] = (
                pltpu.async_copy(s0_hbm.at[dst_v[b]], s0_g[b], sems[b][0]),
                pltpu.async_copy(s1_hbm.at[dst_v[b]], s1_g[b], sems[b][1]),
            )
        if k > 0:
            b = (k - 1) % 2
            off = base + (k - 1) * C
            copies[b][0].wait()
            copies[b][1].wait()
            s0g = s0_g[b]
            s1g = s1_g[b]
            pv = p_v[b]

            @plsc.parallel_loop(0, C // 2, unroll=8)
            def _(q):
                r0 = 2 * q
                r1 = r0 + 1
                sA = s0g[r0] + s1g[r0]
                sB = s0g[r1] + s1g[r1]
                s_pair = jnp.where(lo, sA, sB)
                o_v[q] = pv[pl.ds(q * 16, 16)] / s_pair

            pltpu.sync_copy(o_v, a_hbm.at[pl.ds(off // 2, C // 2)])


_MESH = plsc.VectorSubcoreMesh(core_axis_name="c", subcore_axis_name="s")

_SC_PARAMS = pltpu.CompilerParams(use_tc_tiling_on_sc=False)

_pass1 = functools.partial(
    pl.kernel,
    compiler_params=_SC_PARAMS,
    out_type=(
        jax.ShapeDtypeStruct((E * H,), jnp.float32),
        jax.ShapeDtypeStruct((N, 16), jnp.float32),
        jax.ShapeDtypeStruct((N, 16), jnp.float32),
    ),
    mesh=_MESH,
    scratch_types=[
        pltpu.VMEM((C,), jnp.int32),
        pltpu.VMEM((C,), jnp.int32),
        pltpu.VMEM((C,), jnp.int32),
        pltpu.VMEM((C,), jnp.int32),
        pltpu.VMEM((C, 16), jnp.float32),
        pltpu.VMEM((C, 16), jnp.float32),
        pltpu.VMEM((C, 16), jnp.float32),
        pltpu.VMEM((C, 16), jnp.float32),
        pltpu.VMEM((C, 16), jnp.float32),
        pltpu.VMEM((C * H,), jnp.float32),
        pltpu.VMEM_SHARED((N, 16), jnp.float32),
        pltpu.SemaphoreType.DMA,
        pltpu.SemaphoreType.DMA,
        pltpu.SemaphoreType.DMA,
        pltpu.SemaphoreType.DMA,
    ],
)(_sc_pass1)

_pass2 = functools.partial(
    pl.kernel,
    compiler_params=_SC_PARAMS,
    out_type=jax.ShapeDtypeStruct((E // 2, 16), jnp.float32),
    mesh=_MESH,
    scratch_types=[
        pltpu.VMEM((C,), jnp.int32),
        pltpu.VMEM((C,), jnp.int32),
        pltpu.VMEM((C * H,), jnp.float32),
        pltpu.VMEM((C * H,), jnp.float32),
        pltpu.VMEM((C, 16), jnp.float32),
        pltpu.VMEM((C, 16), jnp.float32),
        pltpu.VMEM((C, 16), jnp.float32),
        pltpu.VMEM((C, 16), jnp.float32),
        pltpu.VMEM((C // 2, 16), jnp.float32),
        pltpu.SemaphoreType.DMA,
        pltpu.SemaphoreType.DMA,
        pltpu.SemaphoreType.DMA,
        pltpu.SemaphoreType.DMA,
    ],
)(_sc_pass2)


def kernel(x, edge_index, etype, W, attn_l, attn_r):
    alsum = attn_l.sum(axis=-1)
    arsum = attn_r.sum(axis=-1)
    W4 = W.reshape(T, IN, H, D)
    Wl = jnp.einsum('tihk,tk->tih', W4, alsum)
    Wr = jnp.einsum('tihk,tk->tih', W4, arsum)
    wcat = jnp.concatenate([Wl, Wr], axis=-1)
    wcat = wcat.transpose(1, 0, 2).reshape(IN, T * 16)
    tgt = jnp.concatenate([jnp.arange(16) % 8, 8 + jnp.arange(16) % 8])
    g32 = (jnp.arange(T * 16)[:, None] % 16 == tgt[None, :]).astype(jnp.float32)

    el16, er16 = _tc_tables(x, etype, wcat, g32)

    src = edge_index[0]
    dst = edge_index[1]
    p, s0, s1 = _pass1(el16, er16, src, dst)
    return _pass2(p, dst, s0, s1).reshape(E, H, 1)


# speedup vs baseline: 1.0039x; 1.0039x over previous
import functools

import jax
import jax.numpy as jnp
from jax import lax
from jax.experimental import pallas as pl
from jax.experimental.pallas import tpu as pltpu
from jax.experimental.pallas import tpu_sc as plsc

N = 10000
E = 320000
IN = 128
H = 8
D = 8
T = 8

NC = 2
NS = 16
NW = NC * NS
EPW = E // NW
C = 1000
NCHUNK = EPW // C


def _tc_body(x_ref, et_ref, wcat_ref, g32_ref, el_ref, er_ref):
    z = jnp.dot(x_ref[...], wcat_ref[...], preferred_element_type=jnp.float32)
    colt = lax.broadcasted_iota(jnp.int32, z.shape, 1) // 16
    masked = jnp.where(colt == et_ref[...], z, 0.0)
    elr = jnp.dot(masked, g32_ref[...], preferred_element_type=jnp.float32)
    el_ref[...] = elr[:, :16]
    er_ref[...] = elr[:, 16:]


def _tc_tables(x, etype, wcat, g32):
    return pl.pallas_call(
        _tc_body,
        out_shape=(
            jax.ShapeDtypeStruct((N, 16), jnp.float32),
            jax.ShapeDtypeStruct((N, 16), jnp.float32),
        ),
    )(x, etype.reshape(N, 1), wcat, g32)


def _sc_pass1(el16_hbm, er16_hbm, src_hbm, dst_hbm,
              p_hbm, s0_hbm, s1_hbm,
              src_v0, src_v1, dst_v0, dst_v1,
              el_g0, el_g1, er_g0, er_g1,
              p_v, p_lin, s_sh, sem0, sem1, sem2, sem3):
    cid = lax.axis_index("c")
    sid = lax.axis_index("s")
    wid = sid * NC + cid
    base = wid * EPW
    lo = lax.iota(jnp.int32, 16) < 8
    src_v = [src_v0, src_v1]
    dst_v = [dst_v0, dst_v1]
    el_g = [el_g0, el_g1]
    er_g = [er_g0, er_g1]
    sems = [[sem0, sem1], [sem2, sem3]]

    @pl.when(sid == 0)
    def _():
        def zrow(r, _):
            p_v[r] = jnp.zeros((16,), jnp.float32)
            return 0
        lax.fori_loop(0, C, zrow, 0)
        for k in range(N // C):
            pltpu.sync_copy(p_v, s_sh.at[pl.ds(k * C, C)])

    plsc.subcore_barrier()

    copies = [None, None]
    for k in range(NCHUNK + 1):
        if k < NCHUNK:
            b = k % 2
            off = base + k * C
            pltpu.sync_copy(src_hbm.at[pl.ds(off, C)], src_v[b])
            pltpu.sync_copy(dst_hbm.at[pl.ds(off, C)], dst_v[b])
            copies[b] = (
                pltpu.async_copy(el16_hbm.at[src_v[b]], el_g[b], sems[b][0]),
                pltpu.async_copy(er16_hbm.at[dst_v[b]], er_g[b], sems[b][1]),
            )
        if k > 0:
            b = (k - 1) % 2
            off = base + (k - 1) * C
            copies[b][0].wait()
            copies[b][1].wait()
            elg = el_g[b]
            erg = er_g[b]

            @plsc.parallel_loop(0, C // 2, unroll=4)
            def _(q):
                r0 = 2 * q
                r1 = r0 + 1
                v0 = elg[r0] + erg[r0]
                v0 = jnp.where(v0 > 0, v0, 0.2 * v0)
                e0 = jnp.exp(v0)
                v1 = elg[r1] + erg[r1]
                v1 = jnp.where(v1 > 0, v1, 0.2 * v1)
                e1 = jnp.exp(v1)
                p_v[r0] = e0
                p_v[r1] = e1
                p_lin[pl.ds(q * 16, 16)] = jnp.where(lo, e0, e1)

            pltpu.sync_copy(p_lin, p_hbm.at[pl.ds(off * H, C * H)])
            pltpu.sync_copy(p_v, s_sh.at[dst_v[b]], add=True)

    plsc.subcore_barrier()

    @pl.when(jnp.logical_and(sid == 0, cid == 0))
    def _():
        pltpu.sync_copy(s_sh, s0_hbm)

    @pl.when(jnp.logical_and(sid == 0, cid == 1))
    def _():
        pltpu.sync_copy(s_sh, s1_hbm)


def _sc_pass2(p_hbm, dst_hbm, s0_hbm, s1_hbm, a_hbm,
              dst_v0, dst_v1, p_v0, p_v1,
              s0_g0, s0_g1, s1_g0, s1_g1, o_v,
              sem0, sem1, sem2, sem3):
    cid = lax.axis_index("c")
    sid = lax.axis_index("s")
    wid = sid * NC + cid
    base = wid * EPW
    lo = lax.iota(jnp.int32, 16) < 8
    dst_v = [dst_v0, dst_v1]
    p_v = [p_v0, p_v1]
    s0_g = [s0_g0, s0_g1]
    s1_g = [s1_g0, s1_g1]
    sems = [[sem0, sem1], [sem2, sem3]]

    copies = [None, None]
    for k in range(NCHUNK + 1):
        if k < NCHUNK:
            b = k % 2
            off = base + k * C
            pltpu.sync_copy(dst_hbm.at[pl.ds(off, C)], dst_v[b])
            pltpu.sync_copy(p_hbm.at[pl.ds(off * H, C * H)], p_v[b])
            copies[b] = (
                pltpu.async_copy(s0_hbm.at[dst_v[b]], s0_g[b], sems[b][0]),
                pltpu.async_copy(s1_hbm.at[dst_v[b]], s1_g[b], sems[b][1]),
            )
        if k > 0:
            b = (k - 1) % 2
            off = base + (k - 1) * C
            copies[b][0].wait()
            copies[b][1].wait()
            s0g = s0_g[b]
            s1g = s1_g[b]
            pv = p_v[b]

            @plsc.parallel_loop(0, C // 2, unroll=8)
            def _(q):
                r0 = 2 * q
                r1 = r0 + 1
                sA = s0g[r0] + s1g[r0]
                sB = s0g[r1] + s1g[r1]
                s_pair = jnp.where(lo, sA, sB)
                o_v[q] = pv[pl.ds(q * 16, 16)] / s_pair

            pltpu.sync_copy(o_v, a_hbm.at[pl.ds(off // 2, C // 2)])


_MESH = plsc.VectorSubcoreMesh(core_axis_name="c", subcore_axis_name="s")

_SC_PARAMS = pltpu.CompilerParams(use_tc_tiling_on_sc=False)

_pass1 = functools.partial(
    pl.kernel,
    compiler_params=_SC_PARAMS,
    out_type=(
        jax.ShapeDtypeStruct((E * H,), jnp.float32),
        jax.ShapeDtypeStruct((N, 16), jnp.float32),
        jax.ShapeDtypeStruct((N, 16), jnp.float32),
    ),
    mesh=_MESH,
    scratch_types=[
        pltpu.VMEM((C,), jnp.int32),
        pltpu.VMEM((C,), jnp.int32),
        pltpu.VMEM((C,), jnp.int32),
        pltpu.VMEM((C,), jnp.int32),
        pltpu.VMEM((C, 16), jnp.float32),
        pltpu.VMEM((C, 16), jnp.float32),
        pltpu.VMEM((C, 16), jnp.float32),
        pltpu.VMEM((C, 16), jnp.float32),
        pltpu.VMEM((C, 16), jnp.float32),
        pltpu.VMEM((C * H,), jnp.float32),
        pltpu.VMEM_SHARED((N, 16), jnp.float32),
        pltpu.SemaphoreType.DMA,
        pltpu.SemaphoreType.DMA,
        pltpu.SemaphoreType.DMA,
        pltpu.SemaphoreType.DMA,
    ],
)(_sc_pass1)

_pass2 = functools.partial(
    pl.kernel,
    compiler_params=_SC_PARAMS,
    out_type=jax.ShapeDtypeStruct((E // 2, 16), jnp.float32),
    mesh=_MESH,
    scratch_types=[
        pltpu.VMEM((C,), jnp.int32),
        pltpu.VMEM((C,), jnp.int32),
        pltpu.VMEM((C * H,), jnp.float32),
        pltpu.VMEM((C * H,), jnp.float32),
        pltpu.VMEM((C, 16), jnp.float32),
        pltpu.VMEM((C, 16), jnp.float32),
        pltpu.VMEM((C, 16), jnp.float32),
        pltpu.VMEM((C, 16), jnp.float32),
        pltpu.VMEM((C // 2, 16), jnp.float32),
        pltpu.SemaphoreType.DMA,
        pltpu.SemaphoreType.DMA,
        pltpu.SemaphoreType.DMA,
        pltpu.SemaphoreType.DMA,
    ],
)(_sc_pass2)


def kernel(x, edge_index, etype, W, attn_l, attn_r):
    alsum = attn_l.sum(axis=-1)
    arsum = attn_r.sum(axis=-1)
    W4 = W.reshape(T, IN, H, D)
    Wl = jnp.einsum('tihk,tk->tih', W4, alsum)
    Wr = jnp.einsum('tihk,tk->tih', W4, arsum)
    wcat = jnp.concatenate([Wl, Wr], axis=-1)
    wcat = wcat.transpose(1, 0, 2).reshape(IN, T * 16)
    tgt = jnp.concatenate([jnp.arange(16) % 8, 8 + jnp.arange(16) % 8])
    g32 = (jnp.arange(T * 16)[:, None] % 16 == tgt[None, :]).astype(jnp.float32)

    el16, er16 = _tc_tables(x, etype, wcat, g32)

    src = edge_index[0]
    dst = edge_index[1]
    p, s0, s1 = _pass1(el16, er16, src, dst)
    return _pass2(p, dst, s0, s1).reshape(E, H, 1)
